# baseline (speedup 1.0000x reference)
import functools

import jax
import jax.numpy as jnp
from jax import lax
from jax.experimental import pallas as pl
from jax.experimental.pallas import tpu as pltpu
from jax.experimental.pallas import tpu_sc as plsc

NBINS = 256
VMIN = -4.0
VMAX = 4.0
LANES = 16
NUM_CORES = 2
NUM_SUBCORES = 16
NUM_WORKERS = NUM_CORES * NUM_SUBCORES

SCALE = NBINS / (VMAX - VMIN)


@functools.partial(jax.jit, static_argnums=(1, 2))
def _hist(x, B, N):
    rows_per_worker = B // NUM_WORKERS
    groups = rows_per_worker // LANES
    pairs_per_group = LANES // 2
    total_pairs = rows_per_worker // 2

    mesh = plsc.VectorSubcoreMesh(core_axis_name="c", subcore_axis_name="s")

    @functools.partial(
        pl.kernel,
        mesh=mesh,
        out_type=jax.ShapeDtypeStruct((B, NBINS), jnp.float32),
        scratch_types=[
            pltpu.VMEM((2, N), jnp.float32),
            pltpu.VMEM((2, N), jnp.float32),
            pltpu.VMEM((LANES * NBINS,), jnp.float32),
            pltpu.VMEM((LANES, NBINS), jnp.float32),
        ],
        compiler_params=pltpu.CompilerParams(needs_layout_passes=False),
    )
    def k(x_hbm, out_hbm, bufa, bufb, lhist, outbuf):
        wid = lax.axis_index("s") * NUM_CORES + lax.axis_index("c")
        row0 = wid * rows_per_worker

        lane_i = lax.iota(jnp.int32, LANES) * NBINS
        zeros = jnp.zeros((LANES,), jnp.float32)
        invn = jnp.full((LANES,), 1.0 / N, jnp.float32)
        ones_mask = jnp.ones((LANES,), jnp.bool_)

        def body(sem_a, sem_b):
            def start_pair(pair, buf, sem):
                src = x_hbm.at[pl.ds(row0 + pair * 2, 2)]
                pltpu.make_async_copy(src, buf, sem).start()

            def wait_pair(buf, sem):
                src = x_hbm.at[pl.ds(row0, 2)]
                pltpu.make_async_copy(src, buf, sem).wait()

            def process_row(buf, r, out_slot):
                @plsc.parallel_loop(0, N, LANES, unroll=4)
                def _elem(off):
                    v = buf[r, pl.ds(off, LANES)]
                    t = (v + (-VMIN)) * SCALE
                    t = jnp.minimum(jnp.maximum(t, 0.0), NBINS - 1.0)
                    idx = t.astype(jnp.int32) + lane_i
                    plsc.addupdate_scatter(lhist, [idx], invn, mask=ones_mask)

                @plsc.parallel_loop(0, NBINS, LANES)
                def _red(off):
                    acc = lhist[pl.ds(off, LANES)]
                    lhist[pl.ds(off, LANES)] = zeros
                    for l in range(1, LANES):
                        acc = acc + lhist[pl.ds(l * NBINS + off, LANES)]
                        lhist[pl.ds(l * NBINS + off, LANES)] = zeros
                    outbuf[out_slot, pl.ds(off, LANES)] = acc

            @plsc.parallel_loop(0, LANES * NBINS, LANES, unroll=8)
            def _zero(off):
                lhist[pl.ds(off, LANES)] = zeros

            start_pair(0, bufa, sem_a)

            def group_body(g, _):
                def quad_body(i, _):
                    gp = g * pairs_per_group + 2 * i
                    start_pair(gp + 1, bufb, sem_b)
                    wait_pair(bufa, sem_a)
                    process_row(bufa, 0, 4 * i)
                    process_row(bufa, 1, 4 * i + 1)
                    nxt = jnp.minimum(gp + 2, total_pairs - 1)
                    start_pair(nxt, bufa, sem_a)
                    wait_pair(bufb, sem_b)
                    process_row(bufb, 0, 4 * i + 2)
                    process_row(bufb, 1, 4 * i + 3)
                    return 0

                lax.fori_loop(0, LANES // 4, quad_body, 0)
                gbase = row0 + g * LANES
                pltpu.sync_copy(outbuf, out_hbm.at[pl.ds(gbase, LANES), :])
                return 0

            lax.fori_loop(0, groups, group_body, 0)
            wait_pair(bufa, sem_a)

        pl.run_scoped(
            body,
            sem_a=pltpu.SemaphoreType.DMA(()),
            sem_b=pltpu.SemaphoreType.DMA(()),
        )

    return k(x)


def kernel(input):
    B, N = input.shape
    return _hist(input, B, N)


# speedup vs baseline: 1.0708x; 1.0708x over previous
import functools

import jax
import jax.numpy as jnp
from jax import lax
from jax.experimental import pallas as pl
from jax.experimental.pallas import tpu as pltpu
from jax.experimental.pallas import tpu_sc as plsc

NBINS = 256
VMIN = -4.0
VMAX = 4.0
LANES = 16
NUM_CORES = 2
NUM_SUBCORES = 16
NUM_WORKERS = NUM_CORES * NUM_SUBCORES

SCALE = NBINS / (VMAX - VMIN)


@functools.partial(jax.jit, static_argnums=(1, 2))
def _hist(x, B, N):
    rows_per_worker = B // NUM_WORKERS
    groups = rows_per_worker // LANES
    pairs_per_group = LANES // 2
    total_pairs = rows_per_worker // 2

    mesh = plsc.VectorSubcoreMesh(core_axis_name="c", subcore_axis_name="s")

    @functools.partial(
        pl.kernel,
        mesh=mesh,
        out_type=jax.ShapeDtypeStruct((B, NBINS), jnp.float32),
        scratch_types=[
            pltpu.VMEM((2, N), jnp.float32),
            pltpu.VMEM((2, N), jnp.float32),
            pltpu.VMEM((LANES * NBINS,), jnp.float32),
            pltpu.VMEM((LANES, NBINS), jnp.float32),
        ],
        compiler_params=pltpu.CompilerParams(needs_layout_passes=False),
    )
    def k(x_hbm, out_hbm, bufa, bufb, lhist, outbuf):
        wid = lax.axis_index("s") * NUM_CORES + lax.axis_index("c")
        row0 = wid * rows_per_worker

        lane_i = lax.iota(jnp.int32, LANES) * NBINS
        zeros = jnp.zeros((LANES,), jnp.float32)
        invn = jnp.full((LANES,), 1.0 / N, jnp.float32)
        ones_mask = jnp.ones((LANES,), jnp.bool_)

        def body(sem_a, sem_b):
            def start_pair(pair, buf, sem):
                src = x_hbm.at[pl.ds(row0 + pair * 2, 2)]
                pltpu.make_async_copy(src, buf, sem).start()

            def wait_pair(buf, sem):
                src = x_hbm.at[pl.ds(row0, 2)]
                pltpu.make_async_copy(src, buf, sem).wait()

            def process_row(buf, r, out_slot):
                @plsc.parallel_loop(0, N, LANES, unroll=8)
                def _elem(off):
                    v = buf[r, pl.ds(off, LANES)]
                    t = (v + (-VMIN)) * SCALE
                    t = jnp.minimum(jnp.maximum(t, 0.0), NBINS - 1.0)
                    idx = t.astype(jnp.int32) + lane_i
                    plsc.addupdate_scatter(lhist, [idx], invn, mask=ones_mask)

                @plsc.parallel_loop(0, NBINS, LANES)
                def _red(off):
                    acc = lhist[pl.ds(off, LANES)]
                    lhist[pl.ds(off, LANES)] = zeros
                    for l in range(1, LANES):
                        acc = acc + lhist[pl.ds(l * NBINS + off, LANES)]
                        lhist[pl.ds(l * NBINS + off, LANES)] = zeros
                    outbuf[out_slot, pl.ds(off, LANES)] = acc

            @plsc.parallel_loop(0, LANES * NBINS, LANES, unroll=8)
            def _zero(off):
                lhist[pl.ds(off, LANES)] = zeros

            start_pair(0, bufa, sem_a)

            def group_body(g, _):
                def quad_body(i, _):
                    gp = g * pairs_per_group + 2 * i
                    start_pair(gp + 1, bufb, sem_b)
                    wait_pair(bufa, sem_a)
                    process_row(bufa, 0, 4 * i)
                    process_row(bufa, 1, 4 * i + 1)
                    nxt = jnp.minimum(gp + 2, total_pairs - 1)
                    start_pair(nxt, bufa, sem_a)
                    wait_pair(bufb, sem_b)
                    process_row(bufb, 0, 4 * i + 2)
                    process_row(bufb, 1, 4 * i + 3)
                    return 0

                lax.fori_loop(0, LANES // 4, quad_body, 0)
                gbase = row0 + g * LANES
                pltpu.sync_copy(outbuf, out_hbm.at[pl.ds(gbase, LANES), :])
                return 0

            lax.fori_loop(0, groups, group_body, 0)
            wait_pair(bufa, sem_a)

        pl.run_scoped(
            body,
            sem_a=pltpu.SemaphoreType.DMA(()),
            sem_b=pltpu.SemaphoreType.DMA(()),
        )

    return k(x)


def kernel(input):
    B, N = input.shape
    return _hist(input, B, N)


# speedup vs baseline: 1.0723x; 1.0014x over previous
import functools

import jax
import jax.numpy as jnp
from jax import lax
from jax.experimental import pallas as pl
from jax.experimental.pallas import tpu as pltpu
from jax.experimental.pallas import tpu_sc as plsc

NBINS = 256
VMIN = -4.0
VMAX = 4.0
LANES = 16
NUM_CORES = 2
NUM_SUBCORES = 16
NUM_WORKERS = NUM_CORES * NUM_SUBCORES

SCALE = NBINS / (VMAX - VMIN)


@functools.partial(jax.jit, static_argnums=(1, 2))
def _hist(x, B, N):
    rows_per_worker = B // NUM_WORKERS
    groups = rows_per_worker // LANES
    pairs_per_group = LANES // 2
    total_pairs = rows_per_worker // 2

    mesh = plsc.VectorSubcoreMesh(core_axis_name="c", subcore_axis_name="s")

    @functools.partial(
        pl.kernel,
        mesh=mesh,
        out_type=jax.ShapeDtypeStruct((B, NBINS), jnp.float32),
        scratch_types=[
            pltpu.VMEM((2, N), jnp.float32),
            pltpu.VMEM((2, N), jnp.float32),
            pltpu.VMEM((LANES * NBINS,), jnp.float32),
            pltpu.VMEM((LANES, NBINS), jnp.float32),
        ],
        compiler_params=pltpu.CompilerParams(needs_layout_passes=False),
    )
    def k(x_hbm, out_hbm, bufa, bufb, lhist, outbuf):
        wid = lax.axis_index("s") * NUM_CORES + lax.axis_index("c")
        row0 = wid * rows_per_worker

        lane_i = lax.iota(jnp.int32, LANES) * NBINS
        zeros = jnp.zeros((LANES,), jnp.float32)
        invn = jnp.full((LANES,), 1.0 / N, jnp.float32)
        ones_mask = jnp.ones((LANES,), jnp.bool_)

        def body(sem_a, sem_b):
            def start_pair(pair, buf, sem):
                src = x_hbm.at[pl.ds(row0 + pair * 2, 2)]
                pltpu.make_async_copy(src, buf, sem).start()

            def wait_pair(buf, sem):
                src = x_hbm.at[pl.ds(row0, 2)]
                pltpu.make_async_copy(src, buf, sem).wait()

            def process_row(buf, r, out_slot):
                @plsc.parallel_loop(0, N, LANES, unroll=8)
                def _elem(off):
                    v = buf[r, pl.ds(off, LANES)]
                    t = (v + (-VMIN)) * SCALE
                    t = jnp.minimum(jnp.maximum(t, 0.0), NBINS - 1.0)
                    idx = t.astype(jnp.int32) + lane_i
                    plsc.addupdate_scatter(lhist, [idx], invn, mask=ones_mask)

                @plsc.parallel_loop(0, NBINS, LANES, unroll=2)
                def _red(off):
                    acc = lhist[pl.ds(off, LANES)]
                    lhist[pl.ds(off, LANES)] = zeros
                    for l in range(1, LANES):
                        acc = acc + lhist[pl.ds(l * NBINS + off, LANES)]
                        lhist[pl.ds(l * NBINS + off, LANES)] = zeros
                    outbuf[out_slot, pl.ds(off, LANES)] = acc

            @plsc.parallel_loop(0, LANES * NBINS, LANES, unroll=8)
            def _zero(off):
                lhist[pl.ds(off, LANES)] = zeros

            start_pair(0, bufa, sem_a)

            def group_body(g, _):
                def quad_body(i, _):
                    gp = g * pairs_per_group + 2 * i
                    start_pair(gp + 1, bufb, sem_b)
                    wait_pair(bufa, sem_a)
                    process_row(bufa, 0, 4 * i)
                    process_row(bufa, 1, 4 * i + 1)
                    nxt = jnp.minimum(gp + 2, total_pairs - 1)
                    start_pair(nxt, bufa, sem_a)
                    wait_pair(bufb, sem_b)
                    process_row(bufb, 0, 4 * i + 2)
                    process_row(bufb, 1, 4 * i + 3)
                    return 0

                lax.fori_loop(0, LANES // 4, quad_body, 0)
                gbase = row0 + g * LANES
                pltpu.sync_copy(outbuf, out_hbm.at[pl.ds(gbase, LANES), :])
                return 0

            lax.fori_loop(0, groups, group_body, 0)
            wait_pair(bufa, sem_a)

        pl.run_scoped(
            body,
            sem_a=pltpu.SemaphoreType.DMA(()),
            sem_b=pltpu.SemaphoreType.DMA(()),
        )

    return k(x)


def kernel(input):
    B, N = input.shape
    return _hist(input, B, N)
